# Initial kernel scaffold; baseline (speedup 1.0000x reference)
import jax
import jax.numpy as jnp
from jax.experimental import pallas as pl


def kernel(inputs, edge_index, batch, edge_weight, W_tag, b_tag, gn_weight, gn_bias, gn_alpha, W_in, b_in, W_h, b_h, W_out, b_out):
    raise NotImplementedError("write your pallas kernel here")


# speedup vs baseline: 5.5489x; 5.5489x over previous
import functools

import jax
import jax.numpy as jnp
from jax import lax
from jax.experimental import pallas as pl
from jax.experimental.pallas import tpu as pltpu
from jax.experimental.pallas import tpu_sc as plsc

N = 10000
E = 320000
D = 128
H = 128
G = 64
C = 10
K = 3
NBLOCKS = 3
EPS = 1e-5

NC = 2
NS = 16
NW = NC * NS
L = 16

EPW = E // NW
CHUNK = 128
NCH = 79
EPWP = NCH * CHUNK
RPT = 640
NP2 = NS * RPT
RCP = 8
NZC = RPT // RCP

GP = G + 1
RPP = 320
NPAD = NW * RPP

@functools.lru_cache(maxsize=None)
def _mesh():
    return plsc.VectorSubcoreMesh(core_axis_name="c", subcore_axis_name="s",
                                  num_cores=NC, num_subcores=NS)


def _wid():
    return lax.axis_index("s") * NC + lax.axis_index("c")


def _sc_deg_body(col_hbm, ew_hbm, out_hbm, col_v, ew_v, acc_v):
    w = _wid()
    pltpu.sync_copy(col_hbm.at[w], col_v)
    pltpu.sync_copy(ew_hbm.at[w], ew_v)

    def zero(i, carry):
        acc_v[pl.ds(i * L, L)] = jnp.zeros((L,), jnp.float32)
        return carry
    lax.fori_loop(0, N // L, zero, 0)

    def chunk(i, carry):
        def sub(j, c2):
            idx = col_v[i, pl.ds(j * L, L)]
            val = ew_v[i, pl.ds(j * L, L)]
            plsc.addupdate_scatter(acc_v, [idx], val)
            return c2
        return lax.fori_loop(0, CHUNK // L, sub, carry)
    lax.fori_loop(0, NCH, chunk, 0)
    pltpu.sync_copy(acc_v, out_hbm.at[w])


@functools.lru_cache(maxsize=None)
def _sc_deg():
  return pl.kernel(
    _sc_deg_body,
    out_type=jax.ShapeDtypeStruct((NW, N), jnp.float32),
    mesh=_mesh(),
    compiler_params=pltpu.CompilerParams(needs_layout_passes=False),
    scratch_types=[
        pltpu.VMEM((NCH, CHUNK), jnp.int32),
        pltpu.VMEM((NCH, CHUNK), jnp.float32),
        pltpu.VMEM((N,), jnp.float32),
    ],
)


def _sc_norm_body(row_hbm, col_hbm, ew_hbm, dis_hbm, out_hbm,
                  row_v, col_v, ew_v, dis_v, nrm_v):
    w = _wid()
    pltpu.sync_copy(row_hbm.at[w], row_v)
    pltpu.sync_copy(col_hbm.at[w], col_v)
    pltpu.sync_copy(ew_hbm.at[w], ew_v)
    pltpu.sync_copy(dis_hbm, dis_v)

    def chunk(i, carry):
        def sub(j, c2):
            r = row_v[i, pl.ds(j * L, L)]
            cc = col_v[i, pl.ds(j * L, L)]
            wv = ew_v[i, pl.ds(j * L, L)]
            dr = plsc.load_gather(dis_v, [r])
            dc = plsc.load_gather(dis_v, [cc])
            nrm_v[i, pl.ds(j * L, L)] = dr * wv * dc
            return c2
        return lax.fori_loop(0, CHUNK // L, sub, carry)
    lax.fori_loop(0, NCH, chunk, 0)
    pltpu.sync_copy(nrm_v, out_hbm.at[w])


@functools.lru_cache(maxsize=None)
def _sc_norm():
  return pl.kernel(
    _sc_norm_body,
    out_type=jax.ShapeDtypeStruct((NW, NCH, CHUNK), jnp.float32),
    mesh=_mesh(),
    compiler_params=pltpu.CompilerParams(needs_layout_passes=False),
    scratch_types=[
        pltpu.VMEM((NCH, CHUNK), jnp.int32),
        pltpu.VMEM((NCH, CHUNK), jnp.int32),
        pltpu.VMEM((NCH, CHUNK), jnp.float32),
        pltpu.VMEM((N,), jnp.float32),
        pltpu.VMEM((NCH, CHUNK), jnp.float32),
    ],
)


def _sc_spmv_body(h_hbm, row_hbm, col_hbm, nrm_hbm, out_hbm,
                  row_v, col_v, nrm_v, rows_v, zbuf_v, acc_sh, sem):
    c = lax.axis_index("c")
    s = lax.axis_index("s")
    w = s * NC + c
    pltpu.sync_copy(row_hbm.at[w], row_v)
    pltpu.sync_copy(col_hbm.at[w], col_v)
    pltpu.sync_copy(nrm_hbm.at[w], nrm_v)

    def zfill2(i, carry):
        r = i // (D // L)
        q = i % (D // L)
        zbuf_v[r, pl.ds(q * L, L)] = jnp.zeros((L,), jnp.float32)
        return carry
    lax.fori_loop(0, RCP * (D // L), zfill2, 0)

    def zcopy(t, carry):
        pltpu.sync_copy(zbuf_v, acc_sh.at[pl.ds(s * RPT + t * RCP, RCP)])
        return carry
    lax.fori_loop(0, NZC, zcopy, 0)
    plsc.subcore_barrier()

    def chunk(i, carry):
        pltpu.async_copy(h_hbm.at[row_v.at[i]], rows_v, sem).wait()

        def grp(j, c2):
            nv = nrm_v[i, pl.ds(j * L, L)]
            for e2 in range(L):
                scl = nv[e2]
                e = j * L + e2
                for q in range(D // L):
                    rows_v[e, pl.ds(q * L, L)] = (
                        rows_v[e, pl.ds(q * L, L)] * scl)
            return c2
        lax.fori_loop(0, CHUNK // L, grp, 0)
        pltpu.async_copy(rows_v, acc_sh.at[col_v.at[i]], sem, add=True).wait()
        return carry
    lax.fori_loop(0, NCH, chunk, 0)
    plsc.subcore_barrier()
    pltpu.sync_copy(acc_sh.at[pl.ds(s * RPT, RPT)],
                    out_hbm.at[c, pl.ds(s * RPT, RPT)])


@functools.lru_cache(maxsize=None)
def _sc_spmv():
  return pl.kernel(
    _sc_spmv_body,
    out_type=jax.ShapeDtypeStruct((NC, NP2, D), jnp.float32),
    mesh=_mesh(),
    compiler_params=pltpu.CompilerParams(needs_layout_passes=False),
    scratch_types=[
        pltpu.VMEM((NCH, CHUNK), jnp.int32),
        pltpu.VMEM((NCH, CHUNK), jnp.int32),
        pltpu.VMEM((NCH, CHUNK), jnp.float32),
        pltpu.VMEM((CHUNK, D), jnp.float32),
        pltpu.VMEM((RCP, D), jnp.float32),
        pltpu.VMEM_SHARED((NP2, D), jnp.float32),
        pltpu.SemaphoreType.DMA,
    ],
)


def _sc_pool_body(h_hbm, b_hbm, sum_hbm, max_hbm,
                  h_v, b_v, accs_v, accm_v):
    w = _wid()
    base = w * RPP
    pltpu.sync_copy(h_hbm.at[pl.ds(base, RPP)], h_v)
    pltpu.sync_copy(b_hbm.at[pl.ds(base, RPP)], b_v)

    def zinit(i, carry):
        g = i // (D // L)
        q = i % (D // L)
        accs_v[g, pl.ds(q * L, L)] = jnp.zeros((L,), jnp.float32)
        accm_v[g, pl.ds(q * L, L)] = jnp.full((L,), -3e38, jnp.float32)
        return carry
    lax.fori_loop(0, GP * (D // L), zinit, 0)

    def rowgrp(t, carry):
        bg = b_v[pl.ds(t * L, L)]
        for e2 in range(L):
            g = bg[e2]
            r = t * L + e2
            for q in range(D // L):
                x = h_v[r, pl.ds(q * L, L)]
                accs_v[g, pl.ds(q * L, L)] = accs_v[g, pl.ds(q * L, L)] + x
                accm_v[g, pl.ds(q * L, L)] = jnp.maximum(
                    accm_v[g, pl.ds(q * L, L)], x)
        return carry
    lax.fori_loop(0, RPP // L, rowgrp, 0)
    pltpu.sync_copy(accs_v, sum_hbm.at[w])
    pltpu.sync_copy(accm_v, max_hbm.at[w])


@functools.lru_cache(maxsize=None)
def _sc_pool():
  return pl.kernel(
    _sc_pool_body,
    out_type=(jax.ShapeDtypeStruct((NW, GP, D), jnp.float32),
              jax.ShapeDtypeStruct((NW, GP, D), jnp.float32)),
    mesh=_mesh(),
    compiler_params=pltpu.CompilerParams(needs_layout_passes=False),
    scratch_types=[
        pltpu.VMEM((RPP, D), jnp.float32),
        pltpu.VMEM((RPP,), jnp.int32),
        pltpu.VMEM((GP, D), jnp.float32),
        pltpu.VMEM((GP, D), jnp.float32),
    ],
)


def _tc_dis_body(degp_ref, dis_ref):
    deg = jnp.sum(degp_ref[...], axis=0)
    dis_ref[...] = jnp.where(deg > 0.0, lax.rsqrt(jnp.maximum(deg, 1e-30)), 0.0)


def _tc_dis(degp):
    return pl.pallas_call(
        _tc_dis_body,
        out_shape=jax.ShapeDtypeStruct((N,), jnp.float32),
    )(degp)


def _tc_start_body(x_ref, w_ref, b_ref, out_ref):
    out_ref[...] = (jnp.dot(x_ref[...], w_ref[...],
                            preferred_element_type=jnp.float32)
                    + b_ref[...][None, :])


def _tc_start(x, w, b):
    return pl.pallas_call(
        _tc_start_body,
        out_shape=jax.ShapeDtypeStruct((N, H), jnp.float32),
    )(x, w, b)


def _tc_hop_body(p_ref, out_in_ref, w_ref, out_ref, h_ref):
    hcomb = p_ref[0] + p_ref[1]
    h_ref[...] = hcomb
    out_ref[...] = out_in_ref[...] + jnp.dot(
        hcomb, w_ref[...], preferred_element_type=jnp.float32)


def _tc_hop(p, out_in, w):
    return pl.pallas_call(
        _tc_hop_body,
        out_shape=(jax.ShapeDtypeStruct((N, H), jnp.float32),
                   jax.ShapeDtypeStruct((N, H), jnp.float32)),
    )(p, out_in, w)


def _tc_end_body(p_ref, out_in_ref, w_ref, b_ref, gnw_ref, gnb_ref, gna_ref,
                 hprev_ref, batch_ref, out_ref):
    hcomb = p_ref[0] + p_ref[1]
    conv = (out_in_ref[...]
            + jnp.dot(hcomb, w_ref[...], preferred_element_type=jnp.float32)
            + b_ref[...][None, :])
    batch = batch_ref[...]
    onehot = (batch[:, None]
              == lax.broadcasted_iota(jnp.int32, (N, G), 1)).astype(jnp.float32)
    counts = jnp.maximum(jnp.sum(onehot, axis=0), 1.0)
    mean = jnp.dot(onehot.T, conv,
                   preferred_element_type=jnp.float32) / counts[:, None]
    out_c = conv - gna_ref[...][None, :] * jnp.dot(
        onehot, mean, preferred_element_type=jnp.float32)
    var = jnp.dot(onehot.T, out_c * out_c,
                  preferred_element_type=jnp.float32) / counts[:, None]
    rstd = lax.rsqrt(var + EPS)
    normed = (gnw_ref[...][None, :] * out_c
              * jnp.dot(onehot, rstd, preferred_element_type=jnp.float32)
              + gnb_ref[...][None, :])
    out_ref[...] = jnp.maximum(normed + hprev_ref[...], 0.0)


def _tc_end(p, out_in, w, b, gnw, gnb, gna, hprev, batch):
    return pl.pallas_call(
        _tc_end_body,
        out_shape=jax.ShapeDtypeStruct((N, H), jnp.float32),
    )(p, out_in, w, b, gnw, gnb, gna, hprev, batch)


def _tc_head_body(sump_ref, maxp_ref, batch_ref, win_ref, bin_ref,
                  wh_ref, bh_ref, wout_ref, bout_ref, out_ref):
    sums = jnp.sum(sump_ref[...], axis=0)[:G, :]
    maxs = jnp.max(maxp_ref[...], axis=0)[:G, :]
    batch = batch_ref[...]
    onehot = (batch[:, None]
              == lax.broadcasted_iota(jnp.int32, (N, G), 1)).astype(jnp.float32)
    counts = jnp.maximum(jnp.sum(onehot, axis=0), 1.0)
    mean = sums / counts[:, None]
    flat = jnp.concatenate([mean, maxs], axis=1)
    z = jnp.maximum(jnp.dot(flat, win_ref[...],
                            preferred_element_type=jnp.float32)
                    + bin_ref[...][None, :], 0.0)
    z = jnp.maximum(jnp.dot(z, wh_ref[0],
                            preferred_element_type=jnp.float32)
                    + bh_ref[...][0][None, :], 0.0)
    out_ref[...] = (jnp.dot(z, wout_ref[...],
                            preferred_element_type=jnp.float32)
                    + bout_ref[...][None, :])


def _tc_head(sump, maxp, batch, win, bin_, wh, bh, wout, bout):
    return pl.pallas_call(
        _tc_head_body,
        out_shape=jax.ShapeDtypeStruct((G, C), jnp.float32),
    )(sump, maxp, batch, win, bin_, wh, bh, wout, bout)


def kernel(inputs, edge_index, batch, edge_weight, W_tag, b_tag, gn_weight,
           gn_bias, gn_alpha, W_in, b_in, W_h, b_h, W_out, b_out):
    pad = ((0, 0), (0, EPWP - EPW))
    row = jnp.pad(jnp.reshape(edge_index[0], (NW, EPW)), pad).reshape(
        (NW, NCH, CHUNK))
    col = jnp.pad(jnp.reshape(edge_index[1], (NW, EPW)), pad).reshape(
        (NW, NCH, CHUNK))
    ew = jnp.pad(jnp.reshape(edge_weight, (NW, EPW)), pad).reshape(
        (NW, NCH, CHUNK))

    degp = _sc_deg()(col, ew)
    dis = _tc_dis(degp)
    nrm = _sc_norm()(row, col, ew, dis)

    h = inputs
    for blk in range(NBLOCKS):
        out = _tc_start(h, W_tag[blk, 0], b_tag[blk])
        hk = h
        for k in range(1, K + 1):
            p = _sc_spmv()(hk, row, col, nrm)[:, :N, :]
            if k < K:
                out, hk = _tc_hop(p, out, W_tag[blk, k])
            else:
                hprev = h if blk > 0 else jnp.zeros((N, H), jnp.float32)
                h = _tc_end(p, out, W_tag[blk, k], b_tag[blk], gn_weight[blk],
                            gn_bias[blk], gn_alpha[blk], hprev, batch)

    hpad = jnp.concatenate(
        [h, jnp.zeros((NPAD - N, H), jnp.float32)], axis=0)
    bpad = jnp.concatenate(
        [batch, jnp.full((NPAD - N,), G, jnp.int32)], axis=0)
    sump, maxp = _sc_pool()(hpad, bpad)
    return _tc_head(sump, maxp, batch, W_in, b_in, W_h, b_h, W_out, b_out)
